# Initial kernel scaffold; baseline (speedup 1.0000x reference)
import jax
import jax.numpy as jnp
from jax.experimental import pallas as pl


def kernel(x, h, c, is_leaf, edge_index, w_iou, b_iou, u_iou, bu_iou, u_f, b_f):
    raise NotImplementedError("write your pallas kernel here")


# speedup vs baseline: 3.2389x; 3.2389x over previous
import functools

import jax
import jax.numpy as jnp
from jax import lax
from jax.experimental import pallas as pl
from jax.experimental.pallas import tpu as pltpu
from jax.experimental.pallas import tpu_sc as plsc

H = 128
CHUNK = 80
NW = 32
CPW = 80


def _sc_gather(h, c, src2d, num_chunks):
    e = num_chunks * CHUNK
    mesh = plsc.VectorSubcoreMesh(core_axis_name="c", subcore_axis_name="s")

    @functools.partial(
        pl.kernel,
        mesh=mesh,
        out_type=[jax.ShapeDtypeStruct((e, H), jnp.float32),
                  jax.ShapeDtypeStruct((e, H), jnp.float32)],
        scratch_types=[
            pltpu.VMEM((CPW, CHUNK), jnp.int32),
            pltpu.VMEM((CHUNK, H), jnp.float32),
            pltpu.VMEM((CHUNK, H), jnp.float32),
            pltpu.SemaphoreType.DMA,
            pltpu.SemaphoreType.DMA,
        ],
    )
    def k(h_hbm, c_hbm, src_hbm, hg_hbm, cg_hbm, idx_v, hbuf, cbuf, hsem, csem):
        wid = lax.axis_index("s") * 2 + lax.axis_index("c")
        start = pl.multiple_of(CPW * wid, 8)
        pltpu.sync_copy(src_hbm.at[pl.ds(start, CPW)], idx_v)
        nproc = jnp.minimum(CPW, num_chunks - start)

        def body(i, carry):
            rowbase = (start + i) * CHUNK
            gh = pltpu.async_copy(h_hbm.at[idx_v.at[i]], hbuf, hsem)
            gc = pltpu.async_copy(c_hbm.at[idx_v.at[i]], cbuf, csem)
            gh.wait()
            gc.wait()
            pltpu.sync_copy(hbuf, hg_hbm.at[pl.ds(rowbase, CHUNK)])
            pltpu.sync_copy(cbuf, cg_hbm.at[pl.ds(rowbase, CHUNK)])
            return carry

        lax.fori_loop(0, nproc, body, 0)

    return k(h, c, src2d)


def _tc_cell_body(x_ref, hcat_ref, mc_ref, leaf_ref, w_ref, b_ref, u_ref,
                  bu_ref, uf_ref, bf_ref, h_out, c_out):
    hcat = hcat_ref[...]
    f = jax.nn.sigmoid(
        jnp.dot(hcat, uf_ref[...], preferred_element_type=jnp.float32)
        + bf_ref[...])
    fc = f * mc_ref[...]
    cf = fc[:, :H] + fc[:, H:]
    leaf = leaf_ref[...]
    xw = jnp.dot(x_ref[...], w_ref[...],
                 preferred_element_type=jnp.float32) + b_ref[...]
    hu = jnp.dot(hcat, u_ref[...],
                 preferred_element_type=jnp.float32) + bu_ref[...]
    iou = leaf * xw + (1.0 - leaf) * hu
    gi = jax.nn.sigmoid(iou[:, :H])
    go = jax.nn.sigmoid(iou[:, H:2 * H])
    gu = jnp.tanh(iou[:, 2 * H:])
    c_new = gi * gu + cf
    h_out[...] = go * jnp.tanh(c_new)
    c_out[...] = c_new


def _tc_cell(x, hcat, mc, leaf, w_iou, b_iou, u_iou, bu_iou, u_f, b_f, blk):
    n = x.shape[0]
    grid = (n // blk,)
    row = lambda i: (i, 0)
    rep = lambda i: (0, 0)
    return pl.pallas_call(
        _tc_cell_body,
        grid=grid,
        in_specs=[
            pl.BlockSpec((blk, H), row),
            pl.BlockSpec((blk, 2 * H), row),
            pl.BlockSpec((blk, 2 * H), row),
            pl.BlockSpec((blk, 1), row),
            pl.BlockSpec((H, 3 * H), rep),
            pl.BlockSpec((1, 3 * H), rep),
            pl.BlockSpec((2 * H, 3 * H), rep),
            pl.BlockSpec((1, 3 * H), rep),
            pl.BlockSpec((2 * H, 2 * H), rep),
            pl.BlockSpec((1, 2 * H), rep),
        ],
        out_specs=[pl.BlockSpec((blk, H), row), pl.BlockSpec((blk, H), row)],
        out_shape=[jax.ShapeDtypeStruct((n, H), jnp.float32)] * 2,
    )(x, hcat, mc, leaf, w_iou, b_iou, u_iou, bu_iou, u_f, b_f)


def kernel(x, h, c, is_leaf, edge_index, w_iou, b_iou, u_iou, bu_iou, u_f, b_f):
    n = x.shape[0]
    e = edge_index.shape[1]
    num_chunks = e // CHUNK
    src_pad = jnp.zeros((NW * CPW * CHUNK,), jnp.int32).at[:e].set(edge_index[0])
    src2d = src_pad.reshape(NW * CPW, CHUNK)
    hg, cg = _sc_gather(h, c, src2d, num_chunks)
    hcat = hg.reshape(n, 2 * H)
    mc = cg.reshape(n, 2 * H)
    leaf = is_leaf.astype(jnp.float32)
    return _tc_cell(x, hcat, mc, leaf,
                    w_iou, b_iou.reshape(1, -1),
                    u_iou, bu_iou.reshape(1, -1),
                    u_f, b_f.reshape(1, -1), blk=1000)


# speedup vs baseline: 4.7920x; 1.4795x over previous
import functools

import jax
import jax.numpy as jnp
from jax import lax
from jax.experimental import pallas as pl
from jax.experimental.pallas import tpu as pltpu
from jax.experimental.pallas import tpu_sc as plsc

H = 128
CHUNK = 80
NW = 32
CPW = 40


def _sc_gather(h, c, se2d, so2d, num_chunks, n):
    mesh = plsc.VectorSubcoreMesh(core_axis_name="c", subcore_axis_name="s")

    @functools.partial(
        pl.kernel,
        mesh=mesh,
        out_type=[jax.ShapeDtypeStruct((n, 2 * H), jnp.float32),
                  jax.ShapeDtypeStruct((n, 2 * H), jnp.float32)],
        scratch_types=[
            pltpu.VMEM((CPW, CHUNK), jnp.int32),
            pltpu.VMEM((CPW, CHUNK), jnp.int32),
            pltpu.VMEM((CHUNK, H), jnp.float32),
            pltpu.VMEM((CHUNK, H), jnp.float32),
            pltpu.VMEM((CHUNK, H), jnp.float32),
            pltpu.VMEM((CHUNK, H), jnp.float32),
            pltpu.SemaphoreType.DMA,
            pltpu.SemaphoreType.DMA,
            pltpu.SemaphoreType.DMA,
            pltpu.SemaphoreType.DMA,
        ],
    )
    def k(h_hbm, c_hbm, se_hbm, so_hbm, hg_hbm, cg_hbm,
          idx_e, idx_o, he, ho, ce, co, s1, s2, s3, s4):
        wid = lax.axis_index("s") * 2 + lax.axis_index("c")
        start = pl.multiple_of(CPW * wid, 8)
        pltpu.sync_copy(se_hbm.at[pl.ds(start, CPW)], idx_e)
        pltpu.sync_copy(so_hbm.at[pl.ds(start, CPW)], idx_o)
        nproc = jnp.minimum(CPW, num_chunks - start)

        def body(i, carry):
            nodebase = (start + i) * CHUNK
            g1 = pltpu.async_copy(h_hbm.at[idx_e.at[i]], he, s1)
            g2 = pltpu.async_copy(h_hbm.at[idx_o.at[i]], ho, s2)
            g3 = pltpu.async_copy(c_hbm.at[idx_e.at[i]], ce, s3)
            g4 = pltpu.async_copy(c_hbm.at[idx_o.at[i]], co, s4)
            g1.wait()
            g2.wait()
            g3.wait()
            g4.wait()
            rows = pl.ds(nodebase, CHUNK)
            pltpu.sync_copy(he, hg_hbm.at[rows, pl.ds(0, H)])
            pltpu.sync_copy(ho, hg_hbm.at[rows, pl.ds(H, H)])
            pltpu.sync_copy(ce, cg_hbm.at[rows, pl.ds(0, H)])
            pltpu.sync_copy(co, cg_hbm.at[rows, pl.ds(H, H)])
            return carry

        lax.fori_loop(0, nproc, body, 0)

    return k(h, c, se2d, so2d)


def _tc_cell_body(x_ref, hcat_ref, mc_ref, leaf_ref, w_ref, b_ref, u_ref,
                  bu_ref, uf_ref, bf_ref, h_out, c_out):
    hcat = hcat_ref[...]
    f = jax.nn.sigmoid(
        jnp.dot(hcat, uf_ref[...], preferred_element_type=jnp.float32)
        + bf_ref[...])
    fc = f * mc_ref[...]
    cf = fc[:, :H] + fc[:, H:]
    leaf = leaf_ref[...]
    xw = jnp.dot(x_ref[...], w_ref[...],
                 preferred_element_type=jnp.float32) + b_ref[...]
    hu = jnp.dot(hcat, u_ref[...],
                 preferred_element_type=jnp.float32) + bu_ref[...]
    iou = leaf * xw + (1.0 - leaf) * hu
    gi = jax.nn.sigmoid(iou[:, :H])
    go = jax.nn.sigmoid(iou[:, H:2 * H])
    gu = jnp.tanh(iou[:, 2 * H:])
    c_new = gi * gu + cf
    h_out[...] = go * jnp.tanh(c_new)
    c_out[...] = c_new


def _tc_cell(x, hcat, mc, leaf, w_iou, b_iou, u_iou, bu_iou, u_f, b_f, blk):
    n = x.shape[0]
    grid = (n // blk,)
    row = lambda i: (i, 0)
    rep = lambda i: (0, 0)
    return pl.pallas_call(
        _tc_cell_body,
        grid=grid,
        in_specs=[
            pl.BlockSpec((blk, H), row),
            pl.BlockSpec((blk, 2 * H), row),
            pl.BlockSpec((blk, 2 * H), row),
            pl.BlockSpec((blk, 1), row),
            pl.BlockSpec((H, 3 * H), rep),
            pl.BlockSpec((1, 3 * H), rep),
            pl.BlockSpec((2 * H, 3 * H), rep),
            pl.BlockSpec((1, 3 * H), rep),
            pl.BlockSpec((2 * H, 2 * H), rep),
            pl.BlockSpec((1, 2 * H), rep),
        ],
        out_specs=[pl.BlockSpec((blk, H), row), pl.BlockSpec((blk, H), row)],
        out_shape=[jax.ShapeDtypeStruct((n, H), jnp.float32)] * 2,
    )(x, hcat, mc, leaf, w_iou, b_iou, u_iou, bu_iou, u_f, b_f)


def kernel(x, h, c, is_leaf, edge_index, w_iou, b_iou, u_iou, bu_iou, u_f, b_f):
    n = x.shape[0]
    num_chunks = n // CHUNK
    npad = NW * CPW * CHUNK
    src = edge_index[0]
    se = jnp.zeros((npad,), jnp.int32).at[:n].set(src[0::2])
    so = jnp.zeros((npad,), jnp.int32).at[:n].set(src[1::2])
    hcat, mc = _sc_gather(h, c, se.reshape(NW * CPW, CHUNK),
                          so.reshape(NW * CPW, CHUNK), num_chunks, n)
    leaf = is_leaf.astype(jnp.float32)
    return _tc_cell(x, hcat, mc, leaf,
                    w_iou, b_iou.reshape(1, -1),
                    u_iou, bu_iou.reshape(1, -1),
                    u_f, b_f.reshape(1, -1), blk=1000)


# speedup vs baseline: 5.0276x; 1.0492x over previous
import functools

import jax
import jax.numpy as jnp
from jax import lax
from jax.experimental import pallas as pl
from jax.experimental.pallas import tpu as pltpu
from jax.experimental.pallas import tpu_sc as plsc

H = 128
CHUNK = 80
NW = 32
CPW = 40


def _sc_gather(h, c, se2d, so2d, num_chunks, n):
    mesh = plsc.VectorSubcoreMesh(core_axis_name="c", subcore_axis_name="s")

    @functools.partial(
        pl.kernel,
        mesh=mesh,
        out_type=[jax.ShapeDtypeStruct((n, 2 * H), jnp.float32),
                  jax.ShapeDtypeStruct((n, 2 * H), jnp.float32)],
        scratch_types=[
            pltpu.VMEM((CPW, CHUNK), jnp.int32),
            pltpu.VMEM((CPW, CHUNK), jnp.int32),
            pltpu.VMEM((2, CHUNK, H), jnp.float32),
            pltpu.VMEM((2, CHUNK, H), jnp.float32),
            pltpu.VMEM((2, CHUNK, H), jnp.float32),
            pltpu.VMEM((2, CHUNK, H), jnp.float32),
            pltpu.SemaphoreType.DMA,
            pltpu.SemaphoreType.DMA,
        ],
    )
    def k(h_hbm, c_hbm, se_hbm, so_hbm, hg_hbm, cg_hbm,
          idx_e, idx_o, he, ho, ce, co, g0, g1):
        wid = lax.axis_index("s") * 2 + lax.axis_index("c")
        start = pl.multiple_of(CPW * wid, 8)
        pltpu.sync_copy(se_hbm.at[pl.ds(start, CPW)], idx_e)
        pltpu.sync_copy(so_hbm.at[pl.ds(start, CPW)], idx_o)
        nproc = jnp.minimum(CPW, num_chunks - start)
        gsem = (g0, g1)

        def issue(j, b):
            pltpu.async_copy(h_hbm.at[idx_e.at[j]], he.at[b], gsem[b])
            pltpu.async_copy(h_hbm.at[idx_o.at[j]], ho.at[b], gsem[b])
            pltpu.async_copy(c_hbm.at[idx_e.at[j]], ce.at[b], gsem[b])
            pltpu.async_copy(c_hbm.at[idx_o.at[j]], co.at[b], gsem[b])

        def drain_store(j, b):
            pltpu.make_async_copy(h_hbm.at[idx_e.at[j]], he.at[b], gsem[b]).wait()
            pltpu.make_async_copy(h_hbm.at[idx_o.at[j]], ho.at[b], gsem[b]).wait()
            pltpu.make_async_copy(c_hbm.at[idx_e.at[j]], ce.at[b], gsem[b]).wait()
            pltpu.make_async_copy(c_hbm.at[idx_o.at[j]], co.at[b], gsem[b]).wait()
            rows = pl.ds((start + j) * CHUNK, CHUNK)
            pltpu.sync_copy(he.at[b], hg_hbm.at[rows, pl.ds(0, H)])
            pltpu.sync_copy(ho.at[b], hg_hbm.at[rows, pl.ds(H, H)])
            pltpu.sync_copy(ce.at[b], cg_hbm.at[rows, pl.ds(0, H)])
            pltpu.sync_copy(co.at[b], cg_hbm.at[rows, pl.ds(H, H)])

        @pl.when(nproc > 0)
        def _():
            issue(0, 0)

        def body(t, carry):
            for phase in (0, 1):
                j = 2 * t + phase
                jn = j + 1

                @pl.when(jn < nproc)
                def _():
                    issue(jn, 1 - phase)

                @pl.when(j < nproc)
                def _():
                    drain_store(j, phase)
            return carry

        lax.fori_loop(0, CPW // 2, body, 0, unroll=False)

    return k(h, c, se2d, so2d)


def _tc_cell_body(x_ref, hcat_ref, mc_ref, leaf_ref, w_ref, b_ref, u_ref,
                  bu_ref, uf_ref, bf_ref, h_out, c_out):
    hcat = hcat_ref[...]
    f = jax.nn.sigmoid(
        jnp.dot(hcat, uf_ref[...], preferred_element_type=jnp.float32)
        + bf_ref[...])
    fc = f * mc_ref[...]
    cf = fc[:, :H] + fc[:, H:]
    leaf = leaf_ref[...]
    xw = jnp.dot(x_ref[...], w_ref[...],
                 preferred_element_type=jnp.float32) + b_ref[...]
    hu = jnp.dot(hcat, u_ref[...],
                 preferred_element_type=jnp.float32) + bu_ref[...]
    iou = leaf * xw + (1.0 - leaf) * hu
    gi = jax.nn.sigmoid(iou[:, :H])
    go = jax.nn.sigmoid(iou[:, H:2 * H])
    gu = jnp.tanh(iou[:, 2 * H:])
    c_new = gi * gu + cf
    h_out[...] = go * jnp.tanh(c_new)
    c_out[...] = c_new


def _tc_cell(x, hcat, mc, leaf, w_iou, b_iou, u_iou, bu_iou, u_f, b_f, blk):
    n = x.shape[0]
    grid = (n // blk,)
    row = lambda i: (i, 0)
    rep = lambda i: (0, 0)
    return pl.pallas_call(
        _tc_cell_body,
        grid=grid,
        in_specs=[
            pl.BlockSpec((blk, H), row),
            pl.BlockSpec((blk, 2 * H), row),
            pl.BlockSpec((blk, 2 * H), row),
            pl.BlockSpec((blk, 1), row),
            pl.BlockSpec((H, 3 * H), rep),
            pl.BlockSpec((1, 3 * H), rep),
            pl.BlockSpec((2 * H, 3 * H), rep),
            pl.BlockSpec((1, 3 * H), rep),
            pl.BlockSpec((2 * H, 2 * H), rep),
            pl.BlockSpec((1, 2 * H), rep),
        ],
        out_specs=[pl.BlockSpec((blk, H), row), pl.BlockSpec((blk, H), row)],
        out_shape=[jax.ShapeDtypeStruct((n, H), jnp.float32)] * 2,
    )(x, hcat, mc, leaf, w_iou, b_iou, u_iou, bu_iou, u_f, b_f)


def kernel(x, h, c, is_leaf, edge_index, w_iou, b_iou, u_iou, bu_iou, u_f, b_f):
    n = x.shape[0]
    num_chunks = n // CHUNK
    npad = NW * CPW * CHUNK
    src = edge_index[0]
    se = jnp.zeros((npad,), jnp.int32).at[:n].set(src[0::2])
    so = jnp.zeros((npad,), jnp.int32).at[:n].set(src[1::2])
    hcat, mc = _sc_gather(h, c, se.reshape(NW * CPW, CHUNK),
                          so.reshape(NW * CPW, CHUNK), num_chunks, n)
    leaf = is_leaf.astype(jnp.float32)
    return _tc_cell(x, hcat, mc, leaf,
                    w_iou, b_iou.reshape(1, -1),
                    u_iou, bu_iou.reshape(1, -1),
                    u_f, b_f.reshape(1, -1), blk=1000)
